# baseline (speedup 1.0000x reference)
import functools

import jax
import jax.numpy as jnp
from jax import lax
from jax.experimental import pallas as pl
from jax.experimental.pallas import tpu as pltpu
from jax.experimental.pallas import tpu_sc as plsc

B, L, D_MODEL = 8, 576, 768
K, D_CODEX, G = 1024, 256, 4
DG = D_CODEX // G
DI = D_MODEL // G
BETA = 0.25

N = B * L
ROWS = N * G
NC, NS, LANES = 2, 16, 16
NW = NC * NS
RPW = ROWS // NW
CH = 48
NCH = RPW // CH


def _stage1(Z_ref, WpreT_ref, gng_ref, gnb_ref, codexT_ref,
            ze_ref, idx_ref):
    Zb = Z_ref[0]
    for g in range(G):
        Zg = Zb[:, g * DI:(g + 1) * DI]
        ze = jnp.dot(Zg, WpreT_ref[g],
                     preferred_element_type=jnp.float32)
        mean = jnp.mean(ze)
        var = jnp.mean((ze - mean) ** 2)
        ze = (ze - mean) * lax.rsqrt(var + 1e-5)
        ze = ze * gng_ref[g][None, :] + gnb_ref[g][None, :]
        ze_ref[0, :, g * DG:(g + 1) * DG] = ze

        dot = jnp.dot(ze, codexT_ref[g],
                      preferred_element_type=jnp.float32)
        zsq = jnp.sum(ze * ze, axis=1, keepdims=True)
        csq = jnp.sum(codexT_ref[g] ** 2, axis=0, keepdims=True)
        d2 = (zsq + csq) - 2.0 * dot
        idx_ref[0, :, g] = jnp.argmin(d2, axis=1)


def _stage3(ze_ref, idx_ref, codex_ref, lng_ref, lnb_ref, WpostT_ref,
            bpost_ref, Zq_ref, loss_ref):
    b = pl.program_id(0)
    zq_parts = []
    loss_part = jnp.float32(0.0)
    kiota = lax.broadcasted_iota(jnp.int32, (L, K), 1)
    for g in range(G):
        idxg = idx_ref[0, :, g].reshape(L, 1)
        probs = (kiota == idxg).astype(jnp.float32)
        zq = jnp.dot(probs, codex_ref[g],
                     preferred_element_type=jnp.float32)
        ze = ze_ref[0, :, g * DG:(g + 1) * DG]
        diff = ze - zq
        loss_part = loss_part + jnp.sum(diff * diff)
        zq_parts.append(zq)

    @pl.when(b == 0)
    def _():
        loss_ref[...] = jnp.zeros((1, 1), jnp.float32)

    loss_ref[...] += jnp.reshape(loss_part, (1, 1)) * (
        (1.0 + BETA) / (B * L * D_CODEX))

    zq_full = jnp.concatenate(zq_parts, axis=1)
    mu = jnp.mean(zq_full, axis=1, keepdims=True)
    v = jnp.mean((zq_full - mu) ** 2, axis=1, keepdims=True)
    y = (zq_full - mu) * lax.rsqrt(v + 1e-5)
    y = y * lng_ref[0][None, :] + lnb_ref[0][None, :]
    Zq_ref[0] = (jnp.dot(y, WpostT_ref[...],
                         preferred_element_type=jnp.float32)
                 + bpost_ref[0][None, :])


def _sc_probs(idx_hbm, zeros_hbm, out_hbm, idx_v, buf0, buf1, sem0, sem1):
    wid = lax.axis_index("s") * NC + lax.axis_index("c")
    base = wid * RPW
    pltpu.sync_copy(idx_hbm.at[pl.ds(base, RPW)], idx_v)
    pltpu.sync_copy(zeros_hbm, buf0)
    pltpu.sync_copy(zeros_hbm, buf1)

    bufs = (buf0, buf1)
    sems = (sem0, sem1)
    ones = jnp.full((LANES,), 1.0, jnp.float32)
    zvec = jnp.zeros((LANES,), jnp.float32)
    rowoff = lax.iota(jnp.int32, LANES) * K

    def _dst(j):
        return out_hbm.at[pl.ds((base + j * CH) * K, CH * K)]

    for j in range(NCH):
        buf, sem = bufs[j % 2], sems[j % 2]
        if j >= 2:
            jj = j - 2
            pltpu.make_async_copy(buf, _dst(jj), sem).wait()
            for t in range(CH // LANES):
                old = idx_v[pl.ds(jj * CH + t * LANES, LANES)]
                plsc.store_scatter(buf, [rowoff + (t * LANES * K) + old],
                                   zvec)
        for t in range(CH // LANES):
            cols = idx_v[pl.ds(j * CH + t * LANES, LANES)]
            plsc.store_scatter(buf, [rowoff + (t * LANES * K) + cols],
                               ones)
        pltpu.async_copy(buf, _dst(j), sem)

    for j in (NCH - 2, NCH - 1):
        pltpu.make_async_copy(bufs[j % 2], _dst(j), sems[j % 2]).wait()


def kernel(Z, W_pre, gn_gamma, gn_beta, codex, ln_gamma, ln_beta,
           W_post, b_post):
    WpreT = W_pre.transpose(0, 2, 1)
    codexT = codex.transpose(0, 2, 1)
    WpostT = W_post.T
    gng = gn_gamma.reshape(G, DG)
    gnb = gn_beta.reshape(G, DG)
    lng = ln_gamma.reshape(1, D_CODEX)
    lnb = ln_beta.reshape(1, D_CODEX)
    bp = b_post.reshape(1, D_MODEL)

    full = lambda shape: pl.BlockSpec(shape, lambda b: (0,) * len(shape))

    ze, idx = pl.pallas_call(
        _stage1,
        grid=(B,),
        in_specs=[
            pl.BlockSpec((1, L, D_MODEL), lambda b: (b, 0, 0)),
            full((G, DI, DG)),
            full((G, DG)),
            full((G, DG)),
            full((G, DG, K)),
        ],
        out_specs=[
            pl.BlockSpec((1, L, D_CODEX), lambda b: (b, 0, 0)),
            pl.BlockSpec((1, L, G), lambda b: (b, 0, 0)),
        ],
        out_shape=[
            jax.ShapeDtypeStruct((B, L, D_CODEX), jnp.float32),
            jax.ShapeDtypeStruct((B, L, G), jnp.int32),
        ],
    )(Z, WpreT, gng, gnb, codexT)

    sc_kernel = functools.partial(
        pl.kernel,
        out_type=jax.ShapeDtypeStruct((ROWS * K,), jnp.float32),
        mesh=plsc.VectorSubcoreMesh(core_axis_name="c", subcore_axis_name="s"),
        compiler_params=pltpu.CompilerParams(needs_layout_passes=False),
        scratch_types=[
            pltpu.VMEM((RPW,), jnp.int32),
            pltpu.VMEM((CH * K,), jnp.float32),
            pltpu.VMEM((CH * K,), jnp.float32),
            pltpu.SemaphoreType.DMA,
            pltpu.SemaphoreType.DMA,
        ],
    )(_sc_probs)
    probs = sc_kernel(idx.reshape(ROWS), jnp.zeros((CH * K,), jnp.float32))

    zq_out, loss_out = pl.pallas_call(
        _stage3,
        grid=(B,),
        in_specs=[
            pl.BlockSpec((1, L, D_CODEX), lambda b: (b, 0, 0)),
            pl.BlockSpec((1, L, G), lambda b: (b, 0, 0)),
            full((G, K, DG)),
            full((1, D_CODEX)),
            full((1, D_CODEX)),
            full((D_CODEX, D_MODEL)),
            full((1, D_MODEL)),
        ],
        out_specs=[
            pl.BlockSpec((1, L, D_MODEL), lambda b: (b, 0, 0)),
            pl.BlockSpec((1, 1), lambda b: (0, 0)),
        ],
        out_shape=[
            jax.ShapeDtypeStruct((B, L, D_MODEL), jnp.float32),
            jax.ShapeDtypeStruct((1, 1), jnp.float32),
        ],
    )(ze, idx, codex, lng, lnb, WpostT, bp)

    return (zq_out, loss_out[0, 0], probs.reshape(B, L, G, K))


# speedup vs baseline: 1.8807x; 1.8807x over previous
import jax
import jax.numpy as jnp
from jax import lax
from jax.experimental import pallas as pl
from jax.experimental.pallas import tpu as pltpu

B, L, D_MODEL = 8, 576, 768
K, D_CODEX, G = 1024, 256, 4
DG = D_CODEX // G
DI = D_MODEL // G
BETA = 0.25


def _body(Z_ref, WpreT_ref, gng_ref, gnb_ref, codex_ref, codexT_ref,
          lng_ref, lnb_ref, WpostT_ref, bpost_ref,
          Zq_ref, loss_ref, probs_ref):
    b = pl.program_id(0)
    Zb = Z_ref[0]

    zq_parts = []
    loss_part = jnp.float32(0.0)
    kiota = lax.broadcasted_iota(jnp.int32, (L, K), 1)
    for g in range(G):
        Zg = Zb[:, g * DI:(g + 1) * DI]
        ze = jnp.dot(Zg, WpreT_ref[g],
                     preferred_element_type=jnp.float32)
        mean = jnp.mean(ze)
        var = jnp.mean((ze - mean) ** 2)
        ze = (ze - mean) * lax.rsqrt(var + 1e-5)
        ze = ze * gng_ref[g][None, :] + gnb_ref[g][None, :]

        dot = jnp.dot(ze, codexT_ref[g],
                      preferred_element_type=jnp.float32)
        zsq = jnp.sum(ze * ze, axis=1, keepdims=True)
        csq = jnp.sum(codexT_ref[g] ** 2, axis=0, keepdims=True)
        d2 = (zsq + csq) - 2.0 * dot

        idx = jnp.argmin(d2, axis=1).reshape(L, 1)
        probs = (kiota == idx).astype(jnp.float32)
        probs_ref[0, :, g, :] = probs

        zq = jnp.dot(probs, codex_ref[g],
                     preferred_element_type=jnp.float32)
        diff = ze - zq
        loss_part = loss_part + jnp.sum(diff * diff)
        zq_parts.append(zq)

    @pl.when(b == 0)
    def _():
        loss_ref[...] = jnp.zeros((1, 1), jnp.float32)

    loss_ref[...] += jnp.reshape(loss_part, (1, 1)) * (
        (1.0 + BETA) / (B * L * D_CODEX))

    zq_full = jnp.concatenate(zq_parts, axis=1)
    mu = jnp.mean(zq_full, axis=1, keepdims=True)
    v = jnp.mean((zq_full - mu) ** 2, axis=1, keepdims=True)
    y = (zq_full - mu) * lax.rsqrt(v + 1e-5)
    y = y * lng_ref[0][None, :] + lnb_ref[0][None, :]
    Zq_ref[0] = (jnp.dot(y, WpostT_ref[...],
                         preferred_element_type=jnp.float32)
                 + bpost_ref[0][None, :])


def kernel(Z, W_pre, gn_gamma, gn_beta, codex, ln_gamma, ln_beta,
           W_post, b_post):
    WpreT = W_pre.transpose(0, 2, 1)
    codexT = codex.transpose(0, 2, 1)
    WpostT = W_post.T
    gng = gn_gamma.reshape(G, DG)
    gnb = gn_beta.reshape(G, DG)
    lng = ln_gamma.reshape(1, D_CODEX)
    lnb = ln_beta.reshape(1, D_CODEX)
    bp = b_post.reshape(1, D_MODEL)

    full = lambda shape: pl.BlockSpec(shape, lambda b: (0,) * len(shape))
    zq_out, loss_out, probs_out = pl.pallas_call(
        _body,
        grid=(B,),
        in_specs=[
            pl.BlockSpec((1, L, D_MODEL), lambda b: (b, 0, 0)),
            full((G, DI, DG)),
            full((G, DG)),
            full((G, DG)),
            full((G, K, DG)),
            full((G, DG, K)),
            full((1, D_CODEX)),
            full((1, D_CODEX)),
            full((D_CODEX, D_MODEL)),
            full((1, D_MODEL)),
        ],
        out_specs=[
            pl.BlockSpec((1, L, D_MODEL), lambda b: (b, 0, 0)),
            pl.BlockSpec((1, 1), lambda b: (0, 0)),
            pl.BlockSpec((1, L, G, K), lambda b: (b, 0, 0, 0)),
        ],
        out_shape=[
            jax.ShapeDtypeStruct((B, L, D_MODEL), jnp.float32),
            jax.ShapeDtypeStruct((1, 1), jnp.float32),
            jax.ShapeDtypeStruct((B, L, G, K), jnp.float32),
        ],
    )(Z, WpreT, gng, gnb, codex, codexT, lng, lnb, WpostT, bp)

    return (zq_out, loss_out[0, 0], probs_out)


# speedup vs baseline: 2.0698x; 1.1006x over previous
import jax
import jax.numpy as jnp
from jax import lax
from jax.experimental import pallas as pl
from jax.experimental.pallas import tpu as pltpu

B, L, D_MODEL = 8, 576, 768
K, D_CODEX, G = 1024, 256, 4
DG = D_CODEX // G
DI = D_MODEL // G
BETA = 0.25


def _body(Z_ref, WpreT_ref, gng_ref, gnb_ref, codexb_ref, codexT_ref,
          lng_ref, lnb_ref, WpostT_ref, bpost_ref,
          Zq_ref, loss_ref, probs_ref):
    b = pl.program_id(0)
    Zb = Z_ref[0]

    zq_parts = []
    loss_part = jnp.float32(0.0)
    for g in range(G):
        Zg = Zb[:, g * DI:(g + 1) * DI]
        ze = jnp.dot(Zg, WpreT_ref[g],
                     preferred_element_type=jnp.float32)
        mean = jnp.mean(ze)
        var = jnp.mean((ze - mean) ** 2)
        ze = (ze - mean) * lax.rsqrt(var + 1e-5)
        ze = ze * gng_ref[g][None, :] + gnb_ref[g][None, :]

        dot = jnp.dot(ze, codexT_ref[g],
                      preferred_element_type=jnp.float32)
        zsq = jnp.sum(ze * ze, axis=1, keepdims=True)
        csq = jnp.sum(codexT_ref[g] ** 2, axis=0, keepdims=True)
        d2 = (zsq + csq) - 2.0 * dot

        m = jnp.min(d2, axis=1, keepdims=True)
        probs = (d2 == m).astype(jnp.float32)
        probs_ref[0, :, g, :] = probs

        zq = jnp.dot(probs.astype(jnp.bfloat16), codexb_ref[g],
                     preferred_element_type=jnp.float32)
        diff = ze - zq
        loss_part = loss_part + jnp.sum(diff * diff)
        zq_parts.append(zq)

    @pl.when(b == 0)
    def _():
        loss_ref[...] = jnp.zeros((1, 1), jnp.float32)

    loss_ref[...] += jnp.reshape(loss_part, (1, 1)) * (
        (1.0 + BETA) / (B * L * D_CODEX))

    zq_full = jnp.concatenate(zq_parts, axis=1)
    mu = jnp.mean(zq_full, axis=1, keepdims=True)
    v = jnp.mean((zq_full - mu) ** 2, axis=1, keepdims=True)
    y = (zq_full - mu) * lax.rsqrt(v + 1e-5)
    y = y * lng_ref[0][None, :] + lnb_ref[0][None, :]
    Zq_ref[0] = (jnp.dot(y.astype(jnp.bfloat16), WpostT_ref[...],
                         preferred_element_type=jnp.float32)
                 + bpost_ref[0][None, :])


def kernel(Z, W_pre, gn_gamma, gn_beta, codex, ln_gamma, ln_beta,
           W_post, b_post):
    WpreT = W_pre.transpose(0, 2, 1)
    codexT = codex.transpose(0, 2, 1)
    codexb = codex.astype(jnp.bfloat16)
    WpostT = W_post.T.astype(jnp.bfloat16)
    gng = gn_gamma.reshape(G, DG)
    gnb = gn_beta.reshape(G, DG)
    lng = ln_gamma.reshape(1, D_CODEX)
    lnb = ln_beta.reshape(1, D_CODEX)
    bp = b_post.reshape(1, D_MODEL)

    full = lambda shape: pl.BlockSpec(shape, lambda b: (0,) * len(shape))
    zq_out, loss_out, probs_out = pl.pallas_call(
        _body,
        grid=(B,),
        in_specs=[
            pl.BlockSpec((1, L, D_MODEL), lambda b: (b, 0, 0)),
            full((G, DI, DG)),
            full((G, DG)),
            full((G, DG)),
            full((G, K, DG)),
            full((G, DG, K)),
            full((1, D_CODEX)),
            full((1, D_CODEX)),
            full((D_CODEX, D_MODEL)),
            full((1, D_MODEL)),
        ],
        out_specs=[
            pl.BlockSpec((1, L, D_MODEL), lambda b: (b, 0, 0)),
            pl.BlockSpec((1, 1), lambda b: (0, 0)),
            pl.BlockSpec((1, L, G, K), lambda b: (b, 0, 0, 0)),
        ],
        out_shape=[
            jax.ShapeDtypeStruct((B, L, D_MODEL), jnp.float32),
            jax.ShapeDtypeStruct((1, 1), jnp.float32),
            jax.ShapeDtypeStruct((B, L, G, K), jnp.float32),
        ],
    )(Z, WpreT, gng, gnb, codexb, codexT, lng, lnb, WpostT, bp)

    return (zq_out, loss_out[0, 0], probs_out)
